# baseline (speedup 1.0000x reference)
import functools

import jax
import jax.numpy as jnp
from jax import lax
from jax.experimental import pallas as pl
from jax.experimental.pallas import tpu as pltpu
from jax.experimental.pallas import tpu_sc as plsc

HID = 128
NA = 50000
NB = 100000
NN = 6
NMOL = 2048
AF = 82
BF = 6
DEPTH = 4

NC = 2
NS = 16
NW = NC * NS
LANES = 16


_DN = (((1,), (1,)), ((), ()))
HIDH = HID // 2


def _pack_bf16(y):
    yb = y.astype(jnp.bfloat16)
    lo = lax.bitcast_convert_type(yb[:, :HIDH], jnp.uint16).astype(jnp.uint32)
    hi = lax.bitcast_convert_type(yb[:, HIDH:], jnp.uint16).astype(jnp.uint32)
    return lax.bitcast_convert_type(lo | (hi << 16), jnp.int32)


def _t1_body(x_ref, w_ref, pre_ref, msg_ref):
    acc = lax.dot_general(x_ref[...], w_ref[...], _DN,
                          preferred_element_type=jnp.float32)
    pre_ref[...] = acc
    msg_ref[...] = _pack_bf16(jnp.maximum(acc, 0.0))


def _t1(fbonds, Wi):
    n = fbonds.shape[0]
    blk = 2000
    f = fbonds.shape[1]
    return pl.pallas_call(
        _t1_body,
        grid=(n // blk,),
        in_specs=[pl.BlockSpec((blk, f), lambda i: (i, 0)),
                  pl.BlockSpec((HID, f), lambda i: (0, 0))],
        out_specs=[pl.BlockSpec((blk, HID), lambda i: (i, 0)),
                   pl.BlockSpec((blk, HIDH), lambda i: (i, 0))],
        out_shape=[jax.ShapeDtypeStruct((n, HID), jnp.float32),
                   jax.ShapeDtypeStruct((n, HIDH), jnp.int32)],
    )(fbonds, Wi)


def _t2_body(s_ref, pre_ref, w_ref, msg_ref):
    sb = s_ref[...].astype(jnp.bfloat16)
    acc = lax.dot_general(sb, w_ref[...], _DN,
                          preferred_element_type=jnp.float32)
    msg_ref[...] = _pack_bf16(jnp.maximum(pre_ref[...] + acc, 0.0))


def _t2(S, pre, Whb):
    n = pre.shape[0]
    blk = 2000
    return pl.pallas_call(
        _t2_body,
        grid=(n // blk,),
        in_specs=[pl.BlockSpec((blk, HID), lambda i: (i, 0)),
                  pl.BlockSpec((blk, HID), lambda i: (i, 0)),
                  pl.BlockSpec((HID, HID), lambda i: (0, 0))],
        out_specs=pl.BlockSpec((blk, HIDH), lambda i: (i, 0)),
        out_shape=jax.ShapeDtypeStruct((n, HIDH), jnp.int32),
    )(S, pre, Whb)


def _t3_body(fa_ref, a_ref, wa_ref, wb_ref, bo_ref, out_ref, *, sign):
    acc = lax.dot_general(fa_ref[...], wa_ref[...], _DN,
                          preferred_element_type=jnp.float32)
    acc += lax.dot_general(a_ref[...], wb_ref[...], _DN,
                           preferred_element_type=jnp.float32)
    out_ref[...] = sign * jnp.maximum(acc + bo_ref[...], 0.0)


def _t3(fatoms, A, Wo, bo, sign):
    n = fatoms.shape[0]
    blk = 2000
    WoA = Wo[:, :AF]
    WoB = Wo[:, AF:].astype(jnp.bfloat16)
    bo2 = bo.reshape(1, HID)
    return pl.pallas_call(
        functools.partial(_t3_body, sign=sign),
        grid=(n // blk,),
        in_specs=[pl.BlockSpec((blk, AF), lambda i: (i, 0)),
                  pl.BlockSpec((blk, HID), lambda i: (i, 0)),
                  pl.BlockSpec((HID, AF), lambda i: (0, 0)),
                  pl.BlockSpec((HID, HID), lambda i: (0, 0)),
                  pl.BlockSpec((1, HID), lambda i: (0, 0))],
        out_specs=pl.BlockSpec((blk, HID), lambda i: (i, 0)),
        out_shape=jax.ShapeDtypeStruct((n, HID), jnp.float32),
    )(fatoms, A, WoA, WoB, bo2)


def _t4_body(parts_ref, wh_ref, bh_ref, wo_ref, bo_ref, out_ref):
    rxn = parts_ref[0] + parts_ref[1]
    h = lax.dot_general(rxn, wh_ref[...], _DN,
                        preferred_element_type=jnp.float32)
    h = jnp.maximum(h + bh_ref[...], 0.0)
    o = lax.dot_general(wo_ref[...], h, _DN,
                        preferred_element_type=jnp.float32)
    out_ref[...] = o + bo_ref[0, 0]


def _t4(parts, r_h_W, r_h_b, r_o_W, r_o_b):
    return pl.pallas_call(
        _t4_body,
        in_specs=[pl.BlockSpec((2, NMOL, HID), lambda: (0, 0, 0)),
                  pl.BlockSpec((HID, HID), lambda: (0, 0)),
                  pl.BlockSpec((1, HID), lambda: (0, 0)),
                  pl.BlockSpec((1, HID), lambda: (0, 0)),
                  pl.BlockSpec((1, 1), lambda: (0, 0))],
        out_specs=pl.BlockSpec((1, NMOL), lambda: (0, 0)),
        out_shape=jax.ShapeDtypeStruct((1, NMOL), jnp.float32),
    )(parts, r_h_W, r_h_b.reshape(1, HID), r_o_W,
      r_o_b.reshape(1, 1)).reshape(NMOL, 1)


_CB = 16
_CI = _CB * NN


_NBUF = 4


def _gsum_body(ncw, rw, table_ref, idx_ref, out_ref, idx_v, *bufs):
    g = bufs[0:_NBUF]
    o = bufs[_NBUF:2 * _NBUF]
    sg = bufs[2 * _NBUF:3 * _NBUF]
    so = bufs[3 * _NBUF:4 * _NBUF]
    c = lax.axis_index("c")
    s = lax.axis_index("s")
    wid = s * NC + c
    base_row = pl.multiple_of(wid * rw, 128)
    crow = pl.multiple_of(base_row // _CB, 8)
    pltpu.sync_copy(idx_ref.at[pl.ds(crow, ncw)], idx_v)

    def start_gather(ci, b):
        pltpu.async_copy(table_ref.at[idx_v.at[ci]], g[b], sg[b])

    def out_slice(ci):
        return out_ref.at[pl.ds(pl.multiple_of(base_row + ci * _CB, 8), _CB)]

    def compute(gb, ob):
        def jbody(j, carry):
            r = j * NN
            for col in range(HIDH // LANES):
                sl = pl.ds(col * LANES, LANES)

                def lo(k):
                    return lax.bitcast_convert_type(
                        lax.shift_left(gb[r + k, sl], 16), jnp.float32)

                def hi(k):
                    return lax.bitcast_convert_type(
                        gb[r + k, sl] & jnp.int32(-65536), jnp.float32)
                ob[j, pl.ds(col * LANES, LANES)] = (
                    lo(0) + lo(1) + lo(2) + lo(3) + lo(4) + lo(5))
                ob[j, pl.ds(HIDH + col * LANES, LANES)] = (
                    hi(0) + hi(1) + hi(2) + hi(3) + hi(4) + hi(5))
            return carry
        lax.fori_loop(0, _CB, jbody, 0)

    for b in range(_NBUF):
        start_gather(b, b)

    def ring_body(p, carry):
        for b in range(_NBUF):
            ci = _NBUF * p + b
            pltpu.make_async_copy(table_ref.at[idx_v.at[0]], g[b], sg[b]).wait()

            @pl.when(p > 0)
            def _():
                pltpu.make_async_copy(o[b], out_slice(0), so[b]).wait()
            compute(g[b], o[b])
            pltpu.async_copy(o[b], out_slice(ci), so[b])

            @pl.when(ci + _NBUF < ncw)
            def _():
                start_gather(ci + _NBUF, b)
        return carry

    lax.fori_loop(0, ncw // _NBUF, ring_body, 0)
    for b in range(_NBUF):
        pltpu.make_async_copy(o[b], out_slice(0), so[b]).wait()


def _gsum_pad(nr):
    rw = -(-nr // NW)
    rw = -(-rw // 128) * 128
    return rw, rw * NW


def _gsum(table, idx2d_pad, nr):
    rw, nr_pad = _gsum_pad(nr)
    ncw = rw // _CB
    mesh = plsc.VectorSubcoreMesh(core_axis_name="c", subcore_axis_name="s",
                                  num_cores=NC, num_subcores=NS)
    f = pl.kernel(
        functools.partial(_gsum_body, ncw, rw),
        out_type=jax.ShapeDtypeStruct((nr_pad, HID), jnp.float32),
        mesh=mesh,
        compiler_params=pltpu.CompilerParams(use_tc_tiling_on_sc=False),
        scratch_types=(
            [pltpu.VMEM((ncw, _CI), jnp.int32)]
            + [pltpu.VMEM((_CI, HIDH), jnp.int32)] * _NBUF
            + [pltpu.VMEM((_CB, HID), jnp.float32)] * _NBUF
            + [pltpu.SemaphoreType.DMA] * (2 * _NBUF)
        ),
    )
    return f(table, idx2d_pad)


_SEG_CSZ = 80
_SEG_NCHUNK = NA // _SEG_CSZ
_SEG_CPW = 24
_SEG_PAD = _SEG_CPW * NW


def _seg_body(src_ref, tgt_ref, sscope_ref, tscope_ref, out_ref,
              acc, idx_s, idx_t, av, zb):
    c = lax.axis_index("c")
    s = lax.axis_index("s")
    wid = s * NC + c
    g0 = wid * _SEG_CPW
    nch = jnp.minimum(_SEG_CPW, jnp.maximum(_SEG_NCHUNK - g0, 0))

    for j in range(_CB):
        for col in range(HID // LANES):
            zb[j, pl.ds(col * LANES, LANES)] = jnp.zeros((LANES,), jnp.float32)
    rows_per_sub = NMOL // NS
    for r in range(rows_per_sub // _CB):
        pltpu.sync_copy(
            zb, acc.at[pl.ds(pl.multiple_of(s * rows_per_sub + r * _CB, 8),
                             _CB)])
    plsc.subcore_barrier()

    g0a = pl.multiple_of(g0, 8)
    pltpu.sync_copy(sscope_ref.at[pl.ds(g0a, _SEG_CPW)], idx_s)
    pltpu.sync_copy(tscope_ref.at[pl.ds(g0a, _SEG_CPW)], idx_t)

    def chunk_body(ci, carry):
        base = pl.multiple_of((g0 + ci) * _SEG_CSZ, 8)
        pltpu.sync_copy(tgt_ref.at[pl.ds(base, _SEG_CSZ)], av)
        pltpu.sync_copy(av, acc.at[idx_t.at[ci]], add=True)
        pltpu.sync_copy(src_ref.at[pl.ds(base, _SEG_CSZ)], av)
        pltpu.sync_copy(av, acc.at[idx_s.at[ci]], add=True)
        return carry

    lax.fori_loop(0, nch, chunk_body, 0)
    plsc.subcore_barrier()

    sub0 = pl.multiple_of(s * rows_per_sub, 8)
    pltpu.sync_copy(acc.at[pl.ds(sub0, rows_per_sub)],
                    out_ref.at[c, pl.ds(sub0, rows_per_sub)])


def _seg(src_atom, tgt_atom, sscope2d, tscope2d):
    mesh = plsc.VectorSubcoreMesh(core_axis_name="c", subcore_axis_name="s",
                                  num_cores=NC, num_subcores=NS)
    f = pl.kernel(
        _seg_body,
        out_type=jax.ShapeDtypeStruct((NC, NMOL, HID), jnp.float32),
        mesh=mesh,
        scratch_types=[
            pltpu.VMEM_SHARED((NMOL, HID), jnp.float32),
            pltpu.VMEM((_SEG_CPW, _SEG_CSZ), jnp.int32),
            pltpu.VMEM((_SEG_CPW, _SEG_CSZ), jnp.int32),
            pltpu.VMEM((_SEG_CSZ, HID), jnp.float32),
            pltpu.VMEM((_CB, HID), jnp.float32),
        ],
    )
    return f(src_atom, tgt_atom, sscope2d, tscope2d)


def _pad_idx(idx, nr):
    _, nr_pad = _gsum_pad(nr)
    idx2d = idx.astype(jnp.int32).reshape(nr // _CB, _CI)
    return jnp.pad(idx2d, ((0, (nr_pad - nr) // _CB), (0, 0)))


def _graph_conv(fatoms, fbonds, agraph, bgraph, Wi, Wh, Wo, bo, sign):
    pre, msg = _t1(fbonds, Wi)
    bidx = _pad_idx(bgraph, NB)
    aidx = _pad_idx(agraph, NA)
    Whb = Wh.astype(jnp.bfloat16)
    for _ in range(DEPTH - 1):
        S = _gsum(msg, bidx, NB)
        msg = _t2(S, pre, Whb)
    A = _gsum(msg, aidx, NA)
    return _t3(fatoms, A, Wo, bo, sign)


def _pad_scope(scope):
    s2 = scope.astype(jnp.int32).reshape(_SEG_NCHUNK, _SEG_CSZ)
    return jnp.pad(s2, ((0, _SEG_PAD - _SEG_NCHUNK), (0, 0)))


def kernel(src_fatoms, src_fbonds, src_agraph, src_bgraph, src_scope_ids,
           tgt_fatoms, tgt_fbonds, tgt_agraph, tgt_bgraph, tgt_scope_ids,
           src_Wi, src_Wh, src_Wo, src_bo,
           tgt_Wi, tgt_Wh, tgt_Wo, tgt_bo,
           r_h_W, r_h_b, r_o_W, r_o_b):
    src_atom = _graph_conv(src_fatoms, src_fbonds, src_agraph, src_bgraph,
                           src_Wi, src_Wh, src_Wo, src_bo, -1.0)
    tgt_atom = _graph_conv(tgt_fatoms, tgt_fbonds, tgt_agraph, tgt_bgraph,
                           tgt_Wi, tgt_Wh, tgt_Wo, tgt_bo, 1.0)
    parts = _seg(src_atom, tgt_atom,
                 _pad_scope(src_scope_ids), _pad_scope(tgt_scope_ids))
    return _t4(parts, r_h_W, r_h_b, r_o_W, r_o_b)


# speedup vs baseline: 1.0491x; 1.0491x over previous
import functools

import jax
import jax.numpy as jnp
from jax import lax
from jax.experimental import pallas as pl
from jax.experimental.pallas import tpu as pltpu
from jax.experimental.pallas import tpu_sc as plsc

HID = 128
NA = 50000
NB = 100000
NN = 6
NMOL = 2048
AF = 82
BF = 6
DEPTH = 4

NC = 2
NS = 16
NW = NC * NS
LANES = 16


_DN = (((1,), (1,)), ((), ()))
HIDH = HID // 2


def _pack_bf16(y):
    yb = y.astype(jnp.bfloat16)
    lo = lax.bitcast_convert_type(yb[:, :HIDH], jnp.uint16).astype(jnp.uint32)
    hi = lax.bitcast_convert_type(yb[:, HIDH:], jnp.uint16).astype(jnp.uint32)
    return lax.bitcast_convert_type(lo | (hi << 16), jnp.int32)


def _unpack_bf16(s):
    su = lax.bitcast_convert_type(s, jnp.uint32)
    lo = lax.bitcast_convert_type((su & 0xFFFF).astype(jnp.uint16),
                                  jnp.bfloat16)
    hi = lax.bitcast_convert_type((su >> 16).astype(jnp.uint16), jnp.bfloat16)
    return jnp.concatenate([lo, hi], axis=1)


def _t1_body(x_ref, w_ref, pre_ref, msg_ref):
    acc = lax.dot_general(x_ref[...], w_ref[...], _DN,
                          preferred_element_type=jnp.float32)
    pre_ref[...] = acc
    msg_ref[...] = _pack_bf16(jnp.maximum(acc, 0.0))


def _t1(fbonds, Wi):
    n = fbonds.shape[0]
    blk = 2000
    f = fbonds.shape[1]
    return pl.pallas_call(
        _t1_body,
        grid=(n // blk,),
        in_specs=[pl.BlockSpec((blk, f), lambda i: (i, 0)),
                  pl.BlockSpec((HID, f), lambda i: (0, 0))],
        out_specs=[pl.BlockSpec((blk, HID), lambda i: (i, 0)),
                   pl.BlockSpec((blk, HIDH), lambda i: (i, 0))],
        out_shape=[jax.ShapeDtypeStruct((n, HID), jnp.float32),
                   jax.ShapeDtypeStruct((n, HIDH), jnp.int32)],
    )(fbonds, Wi)


def _t2_body(s_ref, pre_ref, w_ref, msg_ref):
    sb = _unpack_bf16(s_ref[...])
    acc = lax.dot_general(sb, w_ref[...], _DN,
                          preferred_element_type=jnp.float32)
    msg_ref[...] = _pack_bf16(jnp.maximum(pre_ref[...] + acc, 0.0))


def _t2(S, pre, Whb):
    n = pre.shape[0]
    blk = 2000
    return pl.pallas_call(
        _t2_body,
        grid=(n // blk,),
        in_specs=[pl.BlockSpec((blk, HIDH), lambda i: (i, 0)),
                  pl.BlockSpec((blk, HID), lambda i: (i, 0)),
                  pl.BlockSpec((HID, HID), lambda i: (0, 0))],
        out_specs=pl.BlockSpec((blk, HIDH), lambda i: (i, 0)),
        out_shape=jax.ShapeDtypeStruct((n, HIDH), jnp.int32),
    )(S, pre, Whb)


def _t3_body(fa_ref, a_ref, wa_ref, wb_ref, bo_ref, out_ref, *, sign):
    acc = lax.dot_general(fa_ref[...], wa_ref[...], _DN,
                          preferred_element_type=jnp.float32)
    acc += lax.dot_general(_unpack_bf16(a_ref[...]), wb_ref[...], _DN,
                           preferred_element_type=jnp.float32)
    out_ref[...] = sign * jnp.maximum(acc + bo_ref[...], 0.0)


def _t3(fatoms, A, Wo, bo, sign):
    n = fatoms.shape[0]
    blk = 2000
    WoA = Wo[:, :AF]
    WoB = Wo[:, AF:].astype(jnp.bfloat16)
    bo2 = bo.reshape(1, HID)
    return pl.pallas_call(
        functools.partial(_t3_body, sign=sign),
        grid=(n // blk,),
        in_specs=[pl.BlockSpec((blk, AF), lambda i: (i, 0)),
                  pl.BlockSpec((blk, HIDH), lambda i: (i, 0)),
                  pl.BlockSpec((HID, AF), lambda i: (0, 0)),
                  pl.BlockSpec((HID, HID), lambda i: (0, 0)),
                  pl.BlockSpec((1, HID), lambda i: (0, 0))],
        out_specs=pl.BlockSpec((blk, HID), lambda i: (i, 0)),
        out_shape=jax.ShapeDtypeStruct((n, HID), jnp.float32),
    )(fatoms, A, WoA, WoB, bo2)


def _t4_body(parts_ref, wh_ref, bh_ref, wo_ref, bo_ref, out_ref):
    rxn = parts_ref[0] + parts_ref[1]
    h = lax.dot_general(rxn, wh_ref[...], _DN,
                        preferred_element_type=jnp.float32)
    h = jnp.maximum(h + bh_ref[...], 0.0)
    o = lax.dot_general(wo_ref[...], h, _DN,
                        preferred_element_type=jnp.float32)
    out_ref[...] = o + bo_ref[0, 0]


def _t4(parts, r_h_W, r_h_b, r_o_W, r_o_b):
    return pl.pallas_call(
        _t4_body,
        in_specs=[pl.BlockSpec((2, NMOL, HID), lambda: (0, 0, 0)),
                  pl.BlockSpec((HID, HID), lambda: (0, 0)),
                  pl.BlockSpec((1, HID), lambda: (0, 0)),
                  pl.BlockSpec((1, HID), lambda: (0, 0)),
                  pl.BlockSpec((1, 1), lambda: (0, 0))],
        out_specs=pl.BlockSpec((1, NMOL), lambda: (0, 0)),
        out_shape=jax.ShapeDtypeStruct((1, NMOL), jnp.float32),
    )(parts, r_h_W, r_h_b.reshape(1, HID), r_o_W,
      r_o_b.reshape(1, 1)).reshape(NMOL, 1)


_CB = 16
_CI = _CB * NN


_NBUF = 2


def _gsum_body(ncw, rw, table_ref, idx_ref, out_ref, idx_v, *bufs):
    g = bufs[0:_NBUF]
    o = bufs[_NBUF:2 * _NBUF]
    sg = bufs[2 * _NBUF:3 * _NBUF]
    so = bufs[3 * _NBUF:4 * _NBUF]
    c = lax.axis_index("c")
    s = lax.axis_index("s")
    wid = s * NC + c
    base_row = pl.multiple_of(wid * rw, 128)
    crow = pl.multiple_of(base_row // _CB, 8)
    pltpu.sync_copy(idx_ref.at[pl.ds(crow, ncw)], idx_v)

    def start_gather(ci, b):
        pltpu.async_copy(table_ref.at[idx_v.at[ci]], g[b], sg[b])

    def out_slice(ci):
        return out_ref.at[pl.ds(pl.multiple_of(base_row + ci * _CB, 8), _CB)]

    def compute(gb, ob):
        for j in range(_CB):
            r = j * NN
            for col in range(HIDH // LANES):
                sl = pl.ds(col * LANES, LANES)

                def lo(k):
                    return lax.bitcast_convert_type(
                        lax.shift_left(gb[r + k, sl], 16), jnp.float32)

                def hi(k):
                    return lax.bitcast_convert_type(
                        gb[r + k, sl] & jnp.int32(-65536), jnp.float32)
                accl = lo(0) + lo(1) + lo(2) + lo(3) + lo(4) + lo(5)
                acch = hi(0) + hi(1) + hi(2) + hi(3) + hi(4) + hi(5)
                ul = lax.bitcast_convert_type(accl, jnp.int32)
                uh = lax.bitcast_convert_type(acch, jnp.int32)
                lor = lax.shift_right_logical(
                    ul + jnp.int32(0x7FFF)
                    + (lax.shift_right_logical(ul, 16) & jnp.int32(1)), 16)
                hir = (uh + jnp.int32(0x7FFF)
                       + (lax.shift_right_logical(uh, 16) & jnp.int32(1))
                       ) & jnp.int32(-65536)
                ob[j, sl] = lor | hir

    for b in range(_NBUF):
        start_gather(b, b)

    def ring_body(p, carry):
        for b in range(_NBUF):
            ci = _NBUF * p + b
            pltpu.make_async_copy(table_ref.at[idx_v.at[0]], g[b], sg[b]).wait()

            @pl.when(p > 0)
            def _():
                pltpu.make_async_copy(o[b], out_slice(0), so[b]).wait()
            compute(g[b], o[b])
            pltpu.async_copy(o[b], out_slice(ci), so[b])

            @pl.when(ci + _NBUF < ncw)
            def _():
                start_gather(ci + _NBUF, b)
        return carry

    lax.fori_loop(0, ncw // _NBUF, ring_body, 0)
    for b in range(_NBUF):
        pltpu.make_async_copy(o[b], out_slice(0), so[b]).wait()


def _gsum_pad(nr):
    rw = -(-nr // NW)
    rw = -(-rw // 128) * 128
    return rw, rw * NW


def _gsum(table, idx2d_pad, nr):
    rw, nr_pad = _gsum_pad(nr)
    ncw = rw // _CB
    mesh = plsc.VectorSubcoreMesh(core_axis_name="c", subcore_axis_name="s",
                                  num_cores=NC, num_subcores=NS)
    f = pl.kernel(
        functools.partial(_gsum_body, ncw, rw),
        out_type=jax.ShapeDtypeStruct((nr_pad, HIDH), jnp.int32),
        mesh=mesh,
        compiler_params=pltpu.CompilerParams(use_tc_tiling_on_sc=False),
        scratch_types=(
            [pltpu.VMEM((ncw, _CI), jnp.int32)]
            + [pltpu.VMEM((_CI, HIDH), jnp.int32)] * _NBUF
            + [pltpu.VMEM((_CB, HIDH), jnp.int32)] * _NBUF
            + [pltpu.SemaphoreType.DMA] * (2 * _NBUF)
        ),
    )
    return f(table, idx2d_pad)


_SEG_CSZ = 80
_SEG_NCHUNK = NA // _SEG_CSZ
_SEG_CPW = 24
_SEG_PAD = _SEG_CPW * NW


def _seg_body(src_ref, tgt_ref, sscope_ref, tscope_ref, out_ref,
              acc, idx_s, idx_t, av, zb):
    c = lax.axis_index("c")
    s = lax.axis_index("s")
    wid = s * NC + c
    g0 = wid * _SEG_CPW
    nch = jnp.minimum(_SEG_CPW, jnp.maximum(_SEG_NCHUNK - g0, 0))

    for j in range(_CB):
        for col in range(HID // LANES):
            zb[j, pl.ds(col * LANES, LANES)] = jnp.zeros((LANES,), jnp.float32)
    rows_per_sub = NMOL // NS
    for r in range(rows_per_sub // _CB):
        pltpu.sync_copy(
            zb, acc.at[pl.ds(pl.multiple_of(s * rows_per_sub + r * _CB, 8),
                             _CB)])
    plsc.subcore_barrier()

    g0a = pl.multiple_of(g0, 8)
    pltpu.sync_copy(sscope_ref.at[pl.ds(g0a, _SEG_CPW)], idx_s)
    pltpu.sync_copy(tscope_ref.at[pl.ds(g0a, _SEG_CPW)], idx_t)

    def chunk_body(ci, carry):
        base = pl.multiple_of((g0 + ci) * _SEG_CSZ, 8)
        pltpu.sync_copy(tgt_ref.at[pl.ds(base, _SEG_CSZ)], av)
        pltpu.sync_copy(av, acc.at[idx_t.at[ci]], add=True)
        pltpu.sync_copy(src_ref.at[pl.ds(base, _SEG_CSZ)], av)
        pltpu.sync_copy(av, acc.at[idx_s.at[ci]], add=True)
        return carry

    lax.fori_loop(0, nch, chunk_body, 0)
    plsc.subcore_barrier()

    sub0 = pl.multiple_of(s * rows_per_sub, 8)
    pltpu.sync_copy(acc.at[pl.ds(sub0, rows_per_sub)],
                    out_ref.at[c, pl.ds(sub0, rows_per_sub)])


def _seg(src_atom, tgt_atom, sscope2d, tscope2d):
    mesh = plsc.VectorSubcoreMesh(core_axis_name="c", subcore_axis_name="s",
                                  num_cores=NC, num_subcores=NS)
    f = pl.kernel(
        _seg_body,
        out_type=jax.ShapeDtypeStruct((NC, NMOL, HID), jnp.float32),
        mesh=mesh,
        scratch_types=[
            pltpu.VMEM_SHARED((NMOL, HID), jnp.float32),
            pltpu.VMEM((_SEG_CPW, _SEG_CSZ), jnp.int32),
            pltpu.VMEM((_SEG_CPW, _SEG_CSZ), jnp.int32),
            pltpu.VMEM((_SEG_CSZ, HID), jnp.float32),
            pltpu.VMEM((_CB, HID), jnp.float32),
        ],
    )
    return f(src_atom, tgt_atom, sscope2d, tscope2d)


def _pad_idx(idx, nr):
    _, nr_pad = _gsum_pad(nr)
    idx2d = idx.astype(jnp.int32).reshape(nr // _CB, _CI)
    return jnp.pad(idx2d, ((0, (nr_pad - nr) // _CB), (0, 0)))


def _graph_conv(fatoms, fbonds, agraph, bgraph, Wi, Wh, Wo, bo, sign):
    pre, msg = _t1(fbonds, Wi)
    bidx = _pad_idx(bgraph, NB)
    aidx = _pad_idx(agraph, NA)
    Whb = Wh.astype(jnp.bfloat16)
    for _ in range(DEPTH - 1):
        S = _gsum(msg, bidx, NB)
        msg = _t2(S, pre, Whb)
    A = _gsum(msg, aidx, NA)
    return _t3(fatoms, A, Wo, bo, sign)


def _pad_scope(scope):
    s2 = scope.astype(jnp.int32).reshape(_SEG_NCHUNK, _SEG_CSZ)
    return jnp.pad(s2, ((0, _SEG_PAD - _SEG_NCHUNK), (0, 0)))


def kernel(src_fatoms, src_fbonds, src_agraph, src_bgraph, src_scope_ids,
           tgt_fatoms, tgt_fbonds, tgt_agraph, tgt_bgraph, tgt_scope_ids,
           src_Wi, src_Wh, src_Wo, src_bo,
           tgt_Wi, tgt_Wh, tgt_Wo, tgt_bo,
           r_h_W, r_h_b, r_o_W, r_o_b):
    src_atom = _graph_conv(src_fatoms, src_fbonds, src_agraph, src_bgraph,
                           src_Wi, src_Wh, src_Wo, src_bo, -1.0)
    tgt_atom = _graph_conv(tgt_fatoms, tgt_fbonds, tgt_agraph, tgt_bgraph,
                           tgt_Wi, tgt_Wh, tgt_Wo, tgt_bo, 1.0)
    parts = _seg(src_atom, tgt_atom,
                 _pad_scope(src_scope_ids), _pad_scope(tgt_scope_ids))
    return _t4(parts, r_h_W, r_h_b, r_o_W, r_o_b)


# speedup vs baseline: 1.0561x; 1.0067x over previous
import functools

import jax
import jax.numpy as jnp
from jax import lax
from jax.experimental import pallas as pl
from jax.experimental.pallas import tpu as pltpu
from jax.experimental.pallas import tpu_sc as plsc

HID = 128
NA = 50000
NB = 100000
NN = 6
NMOL = 2048
AF = 82
BF = 6
DEPTH = 4

NC = 2
NS = 16
NW = NC * NS
LANES = 16


_DN = (((1,), (1,)), ((), ()))
HIDH = HID // 2


def _pack_bf16(y):
    yb = y.astype(jnp.bfloat16)
    lo = lax.bitcast_convert_type(yb[:, :HIDH], jnp.uint16).astype(jnp.uint32)
    hi = lax.bitcast_convert_type(yb[:, HIDH:], jnp.uint16).astype(jnp.uint32)
    return lax.bitcast_convert_type(lo | (hi << 16), jnp.int32)


def _unpack_bf16(s):
    su = lax.bitcast_convert_type(s, jnp.uint32)
    lo = lax.bitcast_convert_type((su & 0xFFFF).astype(jnp.uint16),
                                  jnp.bfloat16)
    hi = lax.bitcast_convert_type((su >> 16).astype(jnp.uint16), jnp.bfloat16)
    return jnp.concatenate([lo, hi], axis=1)


def _t1_body(x_ref, w_ref, pre_ref, msg_ref):
    acc = lax.dot_general(x_ref[...], w_ref[...], _DN,
                          preferred_element_type=jnp.float32)
    pre_ref[...] = _pack_bf16(acc)
    msg_ref[...] = _pack_bf16(jnp.maximum(acc, 0.0))


def _t1(fbonds, Wi):
    n = fbonds.shape[0]
    blk = 2000
    f = fbonds.shape[1]
    return pl.pallas_call(
        _t1_body,
        grid=(n // blk,),
        in_specs=[pl.BlockSpec((blk, f), lambda i: (i, 0)),
                  pl.BlockSpec((HID, f), lambda i: (0, 0))],
        out_specs=[pl.BlockSpec((blk, HIDH), lambda i: (i, 0)),
                   pl.BlockSpec((blk, HIDH), lambda i: (i, 0))],
        out_shape=[jax.ShapeDtypeStruct((n, HIDH), jnp.int32),
                   jax.ShapeDtypeStruct((n, HIDH), jnp.int32)],
    )(fbonds, Wi)


def _t2_body(s_ref, pre_ref, w_ref, msg_ref):
    sb = _unpack_bf16(s_ref[...])
    acc = lax.dot_general(sb, w_ref[...], _DN,
                          preferred_element_type=jnp.float32)
    pre = _unpack_bf16(pre_ref[...]).astype(jnp.float32)
    msg_ref[...] = _pack_bf16(jnp.maximum(pre + acc, 0.0))


def _t2(S, pre, Whb):
    n = pre.shape[0]
    blk = 2000
    return pl.pallas_call(
        _t2_body,
        grid=(n // blk,),
        in_specs=[pl.BlockSpec((blk, HIDH), lambda i: (i, 0)),
                  pl.BlockSpec((blk, HIDH), lambda i: (i, 0)),
                  pl.BlockSpec((HID, HID), lambda i: (0, 0))],
        out_specs=pl.BlockSpec((blk, HIDH), lambda i: (i, 0)),
        out_shape=jax.ShapeDtypeStruct((n, HIDH), jnp.int32),
    )(S, pre, Whb)


def _t3_body(fa_ref, a_ref, wa_ref, wb_ref, bo_ref, out_ref, *, sign):
    acc = lax.dot_general(fa_ref[...], wa_ref[...], _DN,
                          preferred_element_type=jnp.float32)
    acc += lax.dot_general(_unpack_bf16(a_ref[...]), wb_ref[...], _DN,
                           preferred_element_type=jnp.float32)
    out_ref[...] = sign * jnp.maximum(acc + bo_ref[...], 0.0)


def _t3(fatoms, A, Wo, bo, sign):
    n = fatoms.shape[0]
    blk = 2000
    WoA = Wo[:, :AF]
    WoB = Wo[:, AF:].astype(jnp.bfloat16)
    bo2 = bo.reshape(1, HID)
    return pl.pallas_call(
        functools.partial(_t3_body, sign=sign),
        grid=(n // blk,),
        in_specs=[pl.BlockSpec((blk, AF), lambda i: (i, 0)),
                  pl.BlockSpec((blk, HIDH), lambda i: (i, 0)),
                  pl.BlockSpec((HID, AF), lambda i: (0, 0)),
                  pl.BlockSpec((HID, HID), lambda i: (0, 0)),
                  pl.BlockSpec((1, HID), lambda i: (0, 0))],
        out_specs=pl.BlockSpec((blk, HID), lambda i: (i, 0)),
        out_shape=jax.ShapeDtypeStruct((n, HID), jnp.float32),
    )(fatoms, A, WoA, WoB, bo2)


def _t4_body(parts_ref, wh_ref, bh_ref, wo_ref, bo_ref, out_ref):
    rxn = parts_ref[0] + parts_ref[1]
    h = lax.dot_general(rxn, wh_ref[...], _DN,
                        preferred_element_type=jnp.float32)
    h = jnp.maximum(h + bh_ref[...], 0.0)
    o = lax.dot_general(wo_ref[...], h, _DN,
                        preferred_element_type=jnp.float32)
    out_ref[...] = o + bo_ref[0, 0]


def _t4(parts, r_h_W, r_h_b, r_o_W, r_o_b):
    return pl.pallas_call(
        _t4_body,
        in_specs=[pl.BlockSpec((2, NMOL, HID), lambda: (0, 0, 0)),
                  pl.BlockSpec((HID, HID), lambda: (0, 0)),
                  pl.BlockSpec((1, HID), lambda: (0, 0)),
                  pl.BlockSpec((1, HID), lambda: (0, 0)),
                  pl.BlockSpec((1, 1), lambda: (0, 0))],
        out_specs=pl.BlockSpec((1, NMOL), lambda: (0, 0)),
        out_shape=jax.ShapeDtypeStruct((1, NMOL), jnp.float32),
    )(parts, r_h_W, r_h_b.reshape(1, HID), r_o_W,
      r_o_b.reshape(1, 1)).reshape(NMOL, 1)


_CB = 16
_CI = _CB * NN


_NBUF = 2


def _gsum_body(ncw, rw, table_ref, idx_ref, out_ref, idx_v, *bufs):
    g = bufs[0:_NBUF]
    o = bufs[_NBUF:2 * _NBUF]
    sg = bufs[2 * _NBUF:3 * _NBUF]
    so = bufs[3 * _NBUF:4 * _NBUF]
    c = lax.axis_index("c")
    s = lax.axis_index("s")
    wid = s * NC + c
    base_row = pl.multiple_of(wid * rw, 128)
    crow = pl.multiple_of(base_row // _CB, 8)
    pltpu.sync_copy(idx_ref.at[pl.ds(crow, ncw)], idx_v)

    def start_gather(ci, b):
        pltpu.async_copy(table_ref.at[idx_v.at[ci]], g[b], sg[b])

    def out_slice(ci):
        return out_ref.at[pl.ds(pl.multiple_of(base_row + ci * _CB, 8), _CB)]

    def compute(gb, ob):
        for j in range(_CB):
            r = j * NN
            for col in range(HIDH // LANES):
                sl = pl.ds(col * LANES, LANES)

                def lo(k):
                    return lax.bitcast_convert_type(
                        lax.shift_left(gb[r + k, sl], 16), jnp.float32)

                def hi(k):
                    return lax.bitcast_convert_type(
                        gb[r + k, sl] & jnp.int32(-65536), jnp.float32)
                accl = lo(0) + lo(1) + lo(2) + lo(3) + lo(4) + lo(5)
                acch = hi(0) + hi(1) + hi(2) + hi(3) + hi(4) + hi(5)
                ul = lax.bitcast_convert_type(accl, jnp.int32)
                uh = lax.bitcast_convert_type(acch, jnp.int32)
                lor = lax.shift_right_logical(
                    ul + jnp.int32(0x7FFF)
                    + (lax.shift_right_logical(ul, 16) & jnp.int32(1)), 16)
                hir = (uh + jnp.int32(0x7FFF)
                       + (lax.shift_right_logical(uh, 16) & jnp.int32(1))
                       ) & jnp.int32(-65536)
                ob[j, sl] = lor | hir

    for b in range(_NBUF):
        start_gather(b, b)

    def ring_body(p, carry):
        for b in range(_NBUF):
            ci = _NBUF * p + b
            pltpu.make_async_copy(table_ref.at[idx_v.at[0]], g[b], sg[b]).wait()

            @pl.when(p > 0)
            def _():
                pltpu.make_async_copy(o[b], out_slice(0), so[b]).wait()
            compute(g[b], o[b])
            pltpu.async_copy(o[b], out_slice(ci), so[b])

            @pl.when(ci + _NBUF < ncw)
            def _():
                start_gather(ci + _NBUF, b)
        return carry

    lax.fori_loop(0, ncw // _NBUF, ring_body, 0)
    for b in range(_NBUF):
        pltpu.make_async_copy(o[b], out_slice(0), so[b]).wait()


def _gsum_pad(nr):
    rw = -(-nr // NW)
    rw = -(-rw // 128) * 128
    return rw, rw * NW


def _gsum(table, idx2d_pad, nr):
    rw, nr_pad = _gsum_pad(nr)
    ncw = rw // _CB
    mesh = plsc.VectorSubcoreMesh(core_axis_name="c", subcore_axis_name="s",
                                  num_cores=NC, num_subcores=NS)
    f = pl.kernel(
        functools.partial(_gsum_body, ncw, rw),
        out_type=jax.ShapeDtypeStruct((nr_pad, HIDH), jnp.int32),
        mesh=mesh,
        compiler_params=pltpu.CompilerParams(use_tc_tiling_on_sc=False),
        scratch_types=(
            [pltpu.VMEM((ncw, _CI), jnp.int32)]
            + [pltpu.VMEM((_CI, HIDH), jnp.int32)] * _NBUF
            + [pltpu.VMEM((_CB, HIDH), jnp.int32)] * _NBUF
            + [pltpu.SemaphoreType.DMA] * (2 * _NBUF)
        ),
    )
    return f(table, idx2d_pad)


_SEG_CSZ = 80
_SEG_NCHUNK = NA // _SEG_CSZ
_SEG_CPW = 24
_SEG_PAD = _SEG_CPW * NW


def _seg_body(src_ref, tgt_ref, sscope_ref, tscope_ref, out_ref,
              acc, idx_s, idx_t, zb, avt0, avt1, avs0, avs1,
              st0, st1, ss0, ss1):
    avt = (avt0, avt1)
    avs = (avs0, avs1)
    st = (st0, st1)
    ss = (ss0, ss1)
    c = lax.axis_index("c")
    s = lax.axis_index("s")
    wid = s * NC + c
    g0 = wid * _SEG_CPW
    nch = jnp.minimum(_SEG_CPW, jnp.maximum(_SEG_NCHUNK - g0, 0))

    for j in range(_CB):
        for col in range(HID // LANES):
            zb[j, pl.ds(col * LANES, LANES)] = jnp.zeros((LANES,), jnp.float32)
    rows_per_sub = NMOL // NS
    for r in range(rows_per_sub // _CB):
        pltpu.sync_copy(
            zb, acc.at[pl.ds(pl.multiple_of(s * rows_per_sub + r * _CB, 8),
                             _CB)])
    plsc.subcore_barrier()

    g0a = pl.multiple_of(g0, 8)
    pltpu.sync_copy(sscope_ref.at[pl.ds(g0a, _SEG_CPW)], idx_s)
    pltpu.sync_copy(tscope_ref.at[pl.ds(g0a, _SEG_CPW)], idx_t)

    def rows(ci):
        base = jnp.minimum(g0 + ci, _SEG_NCHUNK - 1) * _SEG_CSZ
        return pl.ds(pl.multiple_of(base, 8), _SEG_CSZ)

    def start(ci, b):
        pltpu.async_copy(tgt_ref.at[rows(ci)], avt[b], st[b])
        pltpu.async_copy(src_ref.at[rows(ci)], avs[b], ss[b])

    def wait(b):
        pltpu.make_async_copy(tgt_ref.at[rows(0)], avt[b], st[b]).wait()
        pltpu.make_async_copy(src_ref.at[rows(0)], avs[b], ss[b]).wait()

    start(0, 0)
    start(1, 1)

    def pair_body(p, carry):
        for b in range(2):
            ci = 2 * p + b
            wait(b)

            @pl.when(ci < nch)
            def _():
                pltpu.sync_copy(avt[b], acc.at[idx_t.at[ci]], add=True)
                pltpu.sync_copy(avs[b], acc.at[idx_s.at[ci]], add=True)

            @pl.when(ci + 2 < _SEG_CPW)
            def _():
                start(ci + 2, b)
        return carry

    lax.fori_loop(0, _SEG_CPW // 2, pair_body, 0)
    plsc.subcore_barrier()

    sub0 = pl.multiple_of(s * rows_per_sub, 8)
    pltpu.sync_copy(acc.at[pl.ds(sub0, rows_per_sub)],
                    out_ref.at[c, pl.ds(sub0, rows_per_sub)])


def _seg(src_atom, tgt_atom, sscope2d, tscope2d):
    mesh = plsc.VectorSubcoreMesh(core_axis_name="c", subcore_axis_name="s",
                                  num_cores=NC, num_subcores=NS)
    f = pl.kernel(
        _seg_body,
        out_type=jax.ShapeDtypeStruct((NC, NMOL, HID), jnp.float32),
        mesh=mesh,
        scratch_types=(
            [pltpu.VMEM_SHARED((NMOL, HID), jnp.float32),
             pltpu.VMEM((_SEG_CPW, _SEG_CSZ), jnp.int32),
             pltpu.VMEM((_SEG_CPW, _SEG_CSZ), jnp.int32),
             pltpu.VMEM((_CB, HID), jnp.float32)]
            + [pltpu.VMEM((_SEG_CSZ, HID), jnp.float32)] * 4
            + [pltpu.SemaphoreType.DMA] * 4
        ),
    )
    return f(src_atom, tgt_atom, sscope2d, tscope2d)


def _pad_idx(idx, nr):
    _, nr_pad = _gsum_pad(nr)
    idx2d = idx.astype(jnp.int32).reshape(nr // _CB, _CI)
    return jnp.pad(idx2d, ((0, (nr_pad - nr) // _CB), (0, 0)))


def _graph_conv(fatoms, fbonds, agraph, bgraph, Wi, Wh, Wo, bo, sign):
    pre, msg = _t1(fbonds, Wi)
    bidx = _pad_idx(bgraph, NB)
    aidx = _pad_idx(agraph, NA)
    Whb = Wh.astype(jnp.bfloat16)
    for _ in range(DEPTH - 1):
        S = _gsum(msg, bidx, NB)
        msg = _t2(S, pre, Whb)
    A = _gsum(msg, aidx, NA)
    return _t3(fatoms, A, Wo, bo, sign)


def _pad_scope(scope):
    s2 = scope.astype(jnp.int32).reshape(_SEG_NCHUNK, _SEG_CSZ)
    return jnp.pad(s2, ((0, _SEG_PAD - _SEG_NCHUNK), (0, 0)))


def kernel(src_fatoms, src_fbonds, src_agraph, src_bgraph, src_scope_ids,
           tgt_fatoms, tgt_fbonds, tgt_agraph, tgt_bgraph, tgt_scope_ids,
           src_Wi, src_Wh, src_Wo, src_bo,
           tgt_Wi, tgt_Wh, tgt_Wo, tgt_bo,
           r_h_W, r_h_b, r_o_W, r_o_b):
    src_atom = _graph_conv(src_fatoms, src_fbonds, src_agraph, src_bgraph,
                           src_Wi, src_Wh, src_Wo, src_bo, -1.0)
    tgt_atom = _graph_conv(tgt_fatoms, tgt_fbonds, tgt_agraph, tgt_bgraph,
                           tgt_Wi, tgt_Wh, tgt_Wo, tgt_bo, 1.0)
    parts = _seg(src_atom, tgt_atom,
                 _pad_scope(src_scope_ids), _pad_scope(tgt_scope_ids))
    return _t4(parts, r_h_W, r_h_b, r_o_W, r_o_b)
